# baseline (speedup 1.0000x reference)
import functools

import jax
import jax.numpy as jnp
from jax import lax
from jax.experimental import pallas as pl
from jax.experimental.pallas import tpu as pltpu
from jax.experimental.pallas import tpu_sc as plsc

B = 16384
D = 64
NC = 2
NS = 16
L = 16
NW = NC * NS
BPW = B // NW
CHUNK = 128
NCHUNK = BPW // CHUNK
ROWS_PER_CHUNK = B // CHUNK


def _sc_body(uidx_hbm, iidx_hbm, wu_hbm, wi_hbm, mu_hbm, bu_hbm, bi_hbm,
             out_hbm, uidx_v, iidx_v, urows_v, irows_v, bu_v, bi_v, mu_v,
             out_v, sem):
    wid = lax.axis_index("s") * NC + lax.axis_index("c")
    base = wid * BPW

    rowbase = wid * NCHUNK
    pltpu.sync_copy(uidx_hbm.at[pl.ds(rowbase, NCHUNK)], uidx_v)
    pltpu.sync_copy(iidx_hbm.at[pl.ds(rowbase, NCHUNK)], iidx_v)
    pltpu.sync_copy(mu_hbm, mu_v.at[pl.ds(0, 1)])

    copies = []
    for k in range(NCHUNK):
        dst = pl.ds(k * CHUNK, CHUNK)
        copies.append(pltpu.async_copy(wu_hbm.at[uidx_v.at[k]],
                                       urows_v.at[dst], sem))
        copies.append(pltpu.async_copy(wi_hbm.at[iidx_v.at[k]],
                                       irows_v.at[dst], sem))
        copies.append(pltpu.async_copy(bu_hbm.at[uidx_v.at[k]],
                                       bu_v.at[dst], sem))
        copies.append(pltpu.async_copy(bi_hbm.at[iidx_v.at[k]],
                                       bi_v.at[dst], sem))
    for c in copies:
        c.wait()

    mu_s = mu_v[pl.ds(0, L)][0]
    lane = lax.iota(jnp.int32, L)

    def group(g, carry):
        row0 = g * L
        rows = row0 + lane
        acc = jnp.zeros((L,), jnp.float32)
        for c in range(D):
            cc = jnp.full((L,), c, jnp.int32)
            u = plsc.load_gather(urows_v, [rows, cc])
            v = plsc.load_gather(irows_v, [rows, cc])
            acc = acc + u * v
        sl = pl.ds(row0, L)
        out_v[sl] = acc + bu_v[sl] + bi_v[sl] + mu_s
        return carry

    lax.fori_loop(0, BPW // L, group, 0)
    pltpu.sync_copy(out_v, out_hbm.at[pl.ds(base, BPW)])


@functools.partial(jax.jit, static_argnames=())
def kernel(x, W_user, W_item, mu, b_user, b_item):
    uidx = x[:, 0].reshape(ROWS_PER_CHUNK, CHUNK)
    iidx = x[:, 1].reshape(ROWS_PER_CHUNK, CHUNK)
    mu_flat = mu.reshape(1)
    bu_flat = b_user.reshape(-1)
    bi_flat = b_item.reshape(-1)

    mesh = plsc.VectorSubcoreMesh(core_axis_name="c", subcore_axis_name="s",
                                  num_cores=NC, num_subcores=NS)
    k = pl.kernel(
        _sc_body,
        out_type=jax.ShapeDtypeStruct((B,), jnp.float32),
        mesh=mesh,
        scratch_types=[
            pltpu.VMEM((NCHUNK, CHUNK), jnp.int32),
            pltpu.VMEM((NCHUNK, CHUNK), jnp.int32),
            pltpu.VMEM((BPW, D), jnp.float32),
            pltpu.VMEM((BPW, D), jnp.float32),
            pltpu.VMEM((BPW,), jnp.float32),
            pltpu.VMEM((BPW,), jnp.float32),
            pltpu.VMEM((L,), jnp.float32),
            pltpu.VMEM((BPW,), jnp.float32),
            pltpu.SemaphoreType.DMA,
        ],
        compiler_params=pltpu.CompilerParams(needs_layout_passes=False,
                                             use_tc_tiling_on_sc=False),
    )
    return k(uidx, iidx, W_user, W_item, mu_flat, bu_flat, bi_flat)
